# Initial kernel scaffold; baseline (speedup 1.0000x reference)
import jax
import jax.numpy as jnp
from jax.experimental import pallas as pl


def kernel(users, posts, post_adjs, up_masking, bn_gamma, bn_beta, W_user_enc, b_user_enc, W_post_enc, b_post_enc, W_gat0, a_src0, a_dst0, W_gat1, a_src1, a_dst1, W_pcls0, b_pcls0, W_pcls1, b_pcls1, W_ucls0, b_ucls0, W_ucls1, b_ucls1):
    raise NotImplementedError("write your pallas kernel here")


# speedup vs baseline: 1.4816x; 1.4816x over previous
import functools

import jax
import jax.numpy as jnp
from jax.experimental import pallas as pl
from jax.experimental.pallas import tpu as pltpu

_N = 50
_F = 128
_D = 32


def _prep_body(users_ref, gamma_ref, beta_ref, Wue_ref, bue_ref, Wpe_ref,
               bpe_ref, Wg0_ref, Wu0_ref, bu0_ref, Wu1_ref, bu1_ref,
               Wp0_ref, bp0_ref, Wp1_ref, bp1_ref,
               mean_ref, ginv_ref, t_ref, vpost_ref, cu_ref, wp_ref, cp_ref,
               Wg0e_ref, bg0_ref):
    u = users_ref[...]
    mean = jnp.mean(u, axis=0, keepdims=True)
    var = jnp.mean((u - mean) * (u - mean), axis=0, keepdims=True)
    ginv = gamma_ref[...] * jax.lax.rsqrt(var + 1e-5)
    mean_ref[...] = mean
    ginv_ref[...] = ginv

    dot = functools.partial(jnp.dot, preferred_element_type=jnp.float32)
    wu = dot(Wu0_ref[...], Wu1_ref[...])
    wu_top = wu[0:_D, :]
    t = dot(Wue_ref[...], wu_top)
    t_ref[...] = t
    vpost_ref[...] = wu[_D:2 * _D, :]
    cu_ref[...] = (dot(bu0_ref[...], Wu1_ref[...]) + bu1_ref[...]
                   + dot(beta_ref[...], t) + dot(bue_ref[...], wu_top))
    wp = dot(Wp0_ref[...], Wp1_ref[...])
    wp_ref[...] = wp
    cp_ref[...] = dot(bp0_ref[...], Wp1_ref[...]) + bp1_ref[...]
    Wg0e_ref[...] = dot(Wpe_ref[...], Wg0_ref[...])
    bg0_ref[...] = dot(bpe_ref[...], Wg0_ref[...])


def _bdot(a, b):
    return jax.lax.dot_general(
        a, b, (((2,), (1,)), ((0,), (0,))),
        preferred_element_type=jnp.float32)


def _attend(hw, adj, a_src, a_dst, cb):
    a_src_b = jnp.broadcast_to(a_src[None], (cb,) + a_src.shape)
    a_dst_b = jnp.broadcast_to(a_dst[None], (cb,) + a_dst.shape)
    es = _bdot(hw, a_src_b)
    ed = jax.lax.dot_general(
        jnp.swapaxes(a_dst_b, 1, 2), hw, (((2,), (2,)), ((0,), (0,))),
        preferred_element_type=jnp.float32)
    e = es + ed
    e = jnp.where(e >= 0, e, 0.2 * e)
    e = jnp.where(adj > 0, e, -1e9)
    m = jnp.max(e, axis=2, keepdims=True)
    p = jnp.exp(e - m)
    s = jnp.sum(p, axis=2, keepdims=True)
    alpha = p / s
    out = _bdot(alpha, hw)
    return jnp.where(out > 0, out, jnp.exp(out) - 1.0)


def _main_body(posts_ref, adj_ref, users_ref, mean_ref, ginv_ref, t_ref,
               vpost_ref, cu_ref, wp_ref, cp_ref, Wg0e_ref, bg0_ref,
               as0_ref, ad0_ref, Wg1_ref, as1_ref, ad1_ref,
               ul_ref, plab_ref):
    cb = posts_ref.shape[0]
    posts = posts_ref[...]
    adj = adj_ref[...]

    Wg0e_b = jnp.broadcast_to(Wg0e_ref[...][None], (cb, _F, _D))
    hw0 = _bdot(posts, Wg0e_b) + bg0_ref[...][None]
    h1 = _attend(hw0, adj, as0_ref[...], ad0_ref[...], cb)

    Wg1_b = jnp.broadcast_to(Wg1_ref[...][None], (cb, _D, _D))
    hw1 = _bdot(h1, Wg1_b)
    pe = _attend(hw1, adj, as1_ref[...], ad1_ref[...], cb)

    wp_b = jnp.broadcast_to(wp_ref[...][None], (cb, _D, 1))
    pco = _bdot(pe, wp_b) + cp_ref[...][None]
    plab_ref[...] = jax.nn.sigmoid(pco)

    mp = jnp.max(pe, axis=1)
    un = (users_ref[...] - mean_ref[...]) * ginv_ref[...]
    uco = (jnp.dot(un, t_ref[...], preferred_element_type=jnp.float32)
           + jnp.dot(mp, vpost_ref[...], preferred_element_type=jnp.float32)
           + cu_ref[...])
    ul_ref[...] = jax.nn.sigmoid(uco)


def kernel(users, posts, post_adjs, up_masking, bn_gamma, bn_beta,
           W_user_enc, b_user_enc, W_post_enc, b_post_enc,
           W_gat0, a_src0, a_dst0, W_gat1, a_src1, a_dst1,
           W_pcls0, b_pcls0, W_pcls1, b_pcls1,
           W_ucls0, b_ucls0, W_ucls1, b_ucls1):
    B, F = users.shape
    N = posts.shape[1]
    D = W_gat0.shape[0]

    row = lambda v: v.reshape(1, -1)
    f32 = jnp.float32

    prep_outs = (
        jax.ShapeDtypeStruct((1, F), f32),
        jax.ShapeDtypeStruct((1, F), f32),
        jax.ShapeDtypeStruct((F, 1), f32),
        jax.ShapeDtypeStruct((D, 1), f32),
        jax.ShapeDtypeStruct((1, 1), f32),
        jax.ShapeDtypeStruct((D, 1), f32),
        jax.ShapeDtypeStruct((1, 1), f32),
        jax.ShapeDtypeStruct((F, D), f32),
        jax.ShapeDtypeStruct((1, D), f32),
    )
    mean, ginv, t, vpost, cu, wp, cp, Wg0e, bg0 = pl.pallas_call(
        _prep_body, out_shape=prep_outs)(
            users, row(bn_gamma), row(bn_beta), W_user_enc, row(b_user_enc),
            W_post_enc, row(b_post_enc), W_gat0, W_ucls0, row(b_ucls0),
            W_ucls1, row(b_ucls1), W_pcls0, row(b_pcls0), W_pcls1,
            row(b_pcls1))

    CB = 64
    grid = (B // CB,)
    full = lambda shape: pl.BlockSpec(shape, lambda i: (0,) * len(shape))
    in_specs = [
        pl.BlockSpec((CB, N, F), lambda i: (i, 0, 0)),
        pl.BlockSpec((CB, N, N), lambda i: (i, 0, 0)),
        pl.BlockSpec((CB, F), lambda i: (i, 0)),
        full((1, F)), full((1, F)), full((F, 1)), full((D, 1)),
        full((1, 1)), full((D, 1)), full((1, 1)), full((F, D)), full((1, D)),
        full((D, 1)), full((D, 1)), full((D, D)), full((D, 1)), full((D, 1)),
    ]
    out_specs = [
        pl.BlockSpec((CB, 1), lambda i: (i, 0)),
        pl.BlockSpec((CB, N, 1), lambda i: (i, 0, 0)),
    ]
    user_label, post_label = pl.pallas_call(
        _main_body,
        grid=grid,
        in_specs=in_specs,
        out_specs=out_specs,
        out_shape=(
            jax.ShapeDtypeStruct((B, 1), f32),
            jax.ShapeDtypeStruct((B, N, 1), f32),
        ),
        compiler_params=pltpu.CompilerParams(
            dimension_semantics=("arbitrary",)),
    )(posts, post_adjs, users, mean, ginv, t, vpost, cu, wp, cp, Wg0e, bg0,
      a_src0.reshape(D, 1), a_dst0.reshape(D, 1), W_gat1,
      a_src1.reshape(D, 1), a_dst1.reshape(D, 1))
    return (user_label, post_label)


# speedup vs baseline: 1.5577x; 1.0514x over previous
import functools

import jax
import jax.numpy as jnp
from jax.experimental import pallas as pl
from jax.experimental.pallas import tpu as pltpu

_N = 50
_F = 128
_D = 32


def _prep_body(users_ref, gamma_ref, beta_ref, Wue_ref, bue_ref, Wpe_ref,
               bpe_ref, Wg0_ref, Wu0_ref, bu0_ref, Wu1_ref, bu1_ref,
               Wp0_ref, bp0_ref, Wp1_ref, bp1_ref,
               mean_ref, ginv_ref, t_ref, vpost_ref, cu_ref, wp_ref, cp_ref,
               Wg0e_ref, bg0_ref):
    u = users_ref[...]
    mean = jnp.mean(u, axis=0, keepdims=True)
    var = jnp.mean((u - mean) * (u - mean), axis=0, keepdims=True)
    ginv = gamma_ref[...] * jax.lax.rsqrt(var + 1e-5)
    mean_ref[...] = mean
    ginv_ref[...] = ginv

    dot = functools.partial(jnp.dot, preferred_element_type=jnp.float32)
    wu = dot(Wu0_ref[...], Wu1_ref[...])
    wu_top = wu[0:_D, :]
    t = dot(Wue_ref[...], wu_top)
    t_ref[...] = t
    vpost_ref[...] = wu[_D:2 * _D, :]
    cu_ref[...] = (dot(bu0_ref[...], Wu1_ref[...]) + bu1_ref[...]
                   + dot(beta_ref[...], t) + dot(bue_ref[...], wu_top))
    wp = dot(Wp0_ref[...], Wp1_ref[...])
    wp_ref[...] = wp
    cp_ref[...] = dot(bp0_ref[...], Wp1_ref[...]) + bp1_ref[...]
    Wg0e_ref[...] = dot(Wpe_ref[...], Wg0_ref[...])
    bg0_ref[...] = dot(bpe_ref[...], Wg0_ref[...])


def _bdot(a, b):
    return jax.lax.dot_general(
        a, b, (((2,), (1,)), ((0,), (0,))),
        preferred_element_type=jnp.float32)


def _attend(hw, adj, a_src, a_dst, cb):
    a_src_b = jnp.broadcast_to(a_src[None], (cb,) + a_src.shape)
    a_dst_b = jnp.broadcast_to(a_dst[None], (cb,) + a_dst.shape)
    es = _bdot(hw, a_src_b)
    ed = jax.lax.dot_general(
        jnp.swapaxes(a_dst_b, 1, 2), hw, (((2,), (2,)), ((0,), (0,))),
        preferred_element_type=jnp.float32)
    e = es + ed
    e = jnp.where(e >= 0, e, 0.2 * e)
    e = jnp.where(adj > 0, e, -1e9)
    m = jnp.max(e, axis=2, keepdims=True)
    p = jnp.exp(e - m)
    s = jnp.sum(p, axis=2, keepdims=True)
    out = _bdot(p, hw) / s
    return jnp.where(out > 0, out, jnp.exp(out) - 1.0)


def _main_body(posts_ref, adj_ref, users_ref, mean_ref, ginv_ref, t_ref,
               vpost_ref, cu_ref, wp_ref, cp_ref, Wg0e_ref, bg0_ref,
               as0_ref, ad0_ref, Wg1_ref, as1_ref, ad1_ref,
               ul_ref, plab_ref):
    cb = posts_ref.shape[0]
    posts = posts_ref[...]
    adj = adj_ref[...]

    Wg0e_b = jnp.broadcast_to(Wg0e_ref[...][None], (cb, _F, _D))
    hw0 = _bdot(posts, Wg0e_b) + bg0_ref[...][None]
    h1 = _attend(hw0, adj, as0_ref[...], ad0_ref[...], cb)

    Wg1_b = jnp.broadcast_to(Wg1_ref[...][None], (cb, _D, _D))
    hw1 = _bdot(h1, Wg1_b)
    pe = _attend(hw1, adj, as1_ref[...], ad1_ref[...], cb)

    wp_b = jnp.broadcast_to(wp_ref[...][None], (cb, _D, 1))
    pco = _bdot(pe, wp_b) + cp_ref[...][None]
    plab_ref[...] = jax.nn.sigmoid(pco)

    mp = jnp.max(pe, axis=1)
    un = (users_ref[...] - mean_ref[...]) * ginv_ref[...]
    uco = (jnp.dot(un, t_ref[...], preferred_element_type=jnp.float32)
           + jnp.dot(mp, vpost_ref[...], preferred_element_type=jnp.float32)
           + cu_ref[...])
    ul_ref[...] = jax.nn.sigmoid(uco)


def kernel(users, posts, post_adjs, up_masking, bn_gamma, bn_beta,
           W_user_enc, b_user_enc, W_post_enc, b_post_enc,
           W_gat0, a_src0, a_dst0, W_gat1, a_src1, a_dst1,
           W_pcls0, b_pcls0, W_pcls1, b_pcls1,
           W_ucls0, b_ucls0, W_ucls1, b_ucls1):
    B, F = users.shape
    N = posts.shape[1]
    D = W_gat0.shape[0]

    row = lambda v: v.reshape(1, -1)
    f32 = jnp.float32

    prep_outs = (
        jax.ShapeDtypeStruct((1, F), f32),
        jax.ShapeDtypeStruct((1, F), f32),
        jax.ShapeDtypeStruct((F, 1), f32),
        jax.ShapeDtypeStruct((D, 1), f32),
        jax.ShapeDtypeStruct((1, 1), f32),
        jax.ShapeDtypeStruct((D, 1), f32),
        jax.ShapeDtypeStruct((1, 1), f32),
        jax.ShapeDtypeStruct((F, D), f32),
        jax.ShapeDtypeStruct((1, D), f32),
    )
    mean, ginv, t, vpost, cu, wp, cp, Wg0e, bg0 = pl.pallas_call(
        _prep_body, out_shape=prep_outs)(
            users, row(bn_gamma), row(bn_beta), W_user_enc, row(b_user_enc),
            W_post_enc, row(b_post_enc), W_gat0, W_ucls0, row(b_ucls0),
            W_ucls1, row(b_ucls1), W_pcls0, row(b_pcls0), W_pcls1,
            row(b_pcls1))

    CB = 64
    grid = (B // CB,)
    full = lambda shape: pl.BlockSpec(shape, lambda i: (0,) * len(shape))
    in_specs = [
        pl.BlockSpec((CB, N, F), lambda i: (i, 0, 0)),
        pl.BlockSpec((CB, N, N), lambda i: (i, 0, 0)),
        pl.BlockSpec((CB, F), lambda i: (i, 0)),
        full((1, F)), full((1, F)), full((F, 1)), full((D, 1)),
        full((1, 1)), full((D, 1)), full((1, 1)), full((F, D)), full((1, D)),
        full((D, 1)), full((D, 1)), full((D, D)), full((D, 1)), full((D, 1)),
    ]
    out_specs = [
        pl.BlockSpec((CB, 1), lambda i: (i, 0)),
        pl.BlockSpec((CB, N, 1), lambda i: (i, 0, 0)),
    ]
    user_label, post_label = pl.pallas_call(
        _main_body,
        grid=grid,
        in_specs=in_specs,
        out_specs=out_specs,
        out_shape=(
            jax.ShapeDtypeStruct((B, 1), f32),
            jax.ShapeDtypeStruct((B, N, 1), f32),
        ),
        compiler_params=pltpu.CompilerParams(
            dimension_semantics=("parallel",)),
    )(posts, post_adjs, users, mean, ginv, t, vpost, cu, wp, cp, Wg0e, bg0,
      a_src0.reshape(D, 1), a_dst0.reshape(D, 1), W_gat1,
      a_src1.reshape(D, 1), a_dst1.reshape(D, 1))
    return (user_label, post_label)
